# baseline (speedup 1.0000x reference)
import jax
import jax.numpy as jnp
from jax.experimental import pallas as pl
from jax.experimental.pallas import tpu as pltpu

_TOTAL_TOK = 16384
_OLD_DIM = 256
_HID_DIM = 512
_NEW_DIM = 128
_TILE = 2048
_NSTEP = _TOTAL_TOK // _TILE
_NSUB = 4
_SUB = _TILE // _NSUB


def _mlp_tile(x_ref, w1_ref, b1_ref, w2_ref, b2_ref, o_hbm, o_vmem, o_sem):
    i = pl.program_id(0)
    slot = jax.lax.rem(i, 2)

    def sub_copy(step, slot, k):
        return pltpu.make_async_copy(
            o_vmem.at[slot, pl.ds(k * _SUB, _SUB), :],
            o_hbm.at[pl.ds(step * _TILE + k * _SUB, _SUB), :],
            o_sem.at[slot],
        )

    @pl.when(i >= 2)
    def _():
        for k in range(_NSUB):
            sub_copy(i - 2, slot, k).wait()

    for k in range(_NSUB):
        x = x_ref[pl.ds(k * _SUB, _SUB), :].astype(jnp.bfloat16)
        h = jax.lax.dot_general(
            x, w1_ref[...].astype(jnp.bfloat16), (((1,), (0,)), ((), ())),
            preferred_element_type=jnp.float32,
        )
        h = jnp.maximum(
            h.astype(jnp.bfloat16) + b1_ref[...].astype(jnp.bfloat16),
            jnp.bfloat16(0.0),
        )
        o = jax.lax.dot_general(
            h, w2_ref[...].astype(jnp.bfloat16), (((1,), (0,)), ((), ())),
            preferred_element_type=jnp.float32,
        )
        o_vmem[slot, pl.ds(k * _SUB, _SUB), :] = o + b2_ref[...]
        sub_copy(i, slot, k).start()

    @pl.when(i == _NSTEP - 1)
    def _():
        for k in range(_NSUB):
            sub_copy(i - 1, 1 - slot, k).wait()
        for k in range(_NSUB):
            sub_copy(i, slot, k).wait()


def kernel(flat, cu_seqlens, W1, b1, W2, b2):
    del cu_seqlens
    b1r = jnp.reshape(b1, (1, _HID_DIM))
    b2r = jnp.reshape(b2, (1, _NEW_DIM))
    out = pl.pallas_call(
        _mlp_tile,
        grid=(_NSTEP,),
        in_specs=[
            pl.BlockSpec((_TILE, _OLD_DIM), lambda i: (i, 0)),
            pl.BlockSpec((_OLD_DIM, _HID_DIM), lambda i: (0, 0)),
            pl.BlockSpec((1, _HID_DIM), lambda i: (0, 0)),
            pl.BlockSpec((_HID_DIM, _NEW_DIM), lambda i: (0, 0)),
            pl.BlockSpec((1, _NEW_DIM), lambda i: (0, 0)),
        ],
        out_specs=pl.BlockSpec(memory_space=pltpu.MemorySpace.HBM),
        out_shape=jax.ShapeDtypeStruct((_TOTAL_TOK, _NEW_DIM), jnp.float32),
        scratch_shapes=[
            pltpu.VMEM((2, _TILE, _NEW_DIM), jnp.float32),
            pltpu.SemaphoreType.DMA((2,)),
        ],
        compiler_params=pltpu.CompilerParams(
            dimension_semantics=("arbitrary",),
        ),
    )(flat, W1, b1r, W2, b2r)
    return out


# speedup vs baseline: 1.0225x; 1.0225x over previous
import jax
import jax.numpy as jnp
from jax.experimental import pallas as pl
from jax.experimental.pallas import tpu as pltpu

_TOTAL_TOK = 16384
_OLD_DIM = 256
_HID_DIM = 512
_NEW_DIM = 128
_GROUP = 2048
_NGROUP = _TOTAL_TOK // _GROUP
_SUB = 512
_NSUB = _GROUP // _SUB


def _mlp_single(x_hbm, w1_ref, b1_ref, w2_ref, b2_ref, o_hbm,
                x_vmem, o_vmem, in_sem, out_sem):
    def in_copy(g):
        return pltpu.make_async_copy(
            x_hbm.at[pl.ds(g * _GROUP, _GROUP), :],
            x_vmem.at[pl.ds(g * _GROUP, _GROUP), :],
            in_sem,
        )

    def out_copy(g):
        return pltpu.make_async_copy(
            o_vmem.at[pl.ds(g * _GROUP, _GROUP), :],
            o_hbm.at[pl.ds(g * _GROUP, _GROUP), :],
            out_sem,
        )

    for g in range(_NGROUP):
        in_copy(g).start()

    for g in range(_NGROUP):
        in_copy(g).wait()
        for k in range(_NSUB):
            r0 = g * _GROUP + k * _SUB
            x = x_vmem[pl.ds(r0, _SUB), :].astype(jnp.bfloat16)
            h = jax.lax.dot_general(
                x, w1_ref[...].astype(jnp.bfloat16), (((1,), (0,)), ((), ())),
                preferred_element_type=jnp.float32,
            )
            h = jnp.maximum(
                h.astype(jnp.bfloat16) + b1_ref[...].astype(jnp.bfloat16),
                jnp.bfloat16(0.0),
            )
            o = jax.lax.dot_general(
                h, w2_ref[...].astype(jnp.bfloat16), (((1,), (0,)), ((), ())),
                preferred_element_type=jnp.float32,
            )
            o_vmem[pl.ds(r0, _SUB), :] = o + b2_ref[...]
        out_copy(g).start()

    for g in range(_NGROUP):
        out_copy(g).wait()


def kernel(flat, cu_seqlens, W1, b1, W2, b2):
    del cu_seqlens
    b1r = jnp.reshape(b1, (1, _HID_DIM))
    b2r = jnp.reshape(b2, (1, _NEW_DIM))
    out = pl.pallas_call(
        _mlp_single,
        in_specs=[
            pl.BlockSpec(memory_space=pltpu.MemorySpace.HBM),
            pl.BlockSpec(memory_space=pltpu.MemorySpace.VMEM),
            pl.BlockSpec(memory_space=pltpu.MemorySpace.VMEM),
            pl.BlockSpec(memory_space=pltpu.MemorySpace.VMEM),
            pl.BlockSpec(memory_space=pltpu.MemorySpace.VMEM),
        ],
        out_specs=pl.BlockSpec(memory_space=pltpu.MemorySpace.HBM),
        out_shape=jax.ShapeDtypeStruct((_TOTAL_TOK, _NEW_DIM), jnp.float32),
        scratch_shapes=[
            pltpu.VMEM((_TOTAL_TOK, _OLD_DIM), jnp.float32),
            pltpu.VMEM((_TOTAL_TOK, _NEW_DIM), jnp.float32),
            pltpu.SemaphoreType.DMA,
            pltpu.SemaphoreType.DMA,
        ],
    )(flat, W1, b1r, W2, b2r)
    return out


# speedup vs baseline: 1.0800x; 1.0562x over previous
import jax
import jax.numpy as jnp
from jax.experimental import pallas as pl
from jax.experimental.pallas import tpu as pltpu

_TOTAL_TOK = 16384
_OLD_DIM = 256
_HID_DIM = 512
_NEW_DIM = 128
_TILE = 4096
_NSTEP = _TOTAL_TOK // _TILE
_NSUB = 8
_SUB = _TILE // _NSUB


def _mlp_tile(x_hbm, w1_ref, b1_ref, w2_ref, b2_ref, o_hbm,
              x_vmem, o_vmem, in_sem, o_sem):
    i = pl.program_id(0)
    slot = jax.lax.rem(i, 2)

    def in_copy(step, slot):
        return pltpu.make_async_copy(
            x_hbm.at[pl.ds(step * _TILE, _TILE), :],
            x_vmem.at[slot],
            in_sem.at[slot],
        )

    def sub_copy(step, slot, k):
        return pltpu.make_async_copy(
            o_vmem.at[slot, pl.ds(k * _SUB, _SUB), :],
            o_hbm.at[pl.ds(step * _TILE + k * _SUB, _SUB), :],
            o_sem.at[slot],
        )

    @pl.when(i == 0)
    def _():
        in_copy(0, 0).start()

    @pl.when(i + 1 < _NSTEP)
    def _():
        in_copy(i + 1, 1 - slot).start()

    @pl.when(i >= 2)
    def _():
        for k in range(_NSUB):
            sub_copy(i - 2, slot, k).wait()

    in_copy(i, slot).wait()

    for k in range(_NSUB):
        x = x_vmem[slot, pl.ds(k * _SUB, _SUB), :].astype(jnp.bfloat16)
        h = jax.lax.dot_general(
            x, w1_ref[...].astype(jnp.bfloat16), (((1,), (0,)), ((), ())),
            preferred_element_type=jnp.float32,
        )
        h = jnp.maximum(
            h.astype(jnp.bfloat16) + b1_ref[...].astype(jnp.bfloat16),
            jnp.bfloat16(0.0),
        )
        o = jax.lax.dot_general(
            h, w2_ref[...].astype(jnp.bfloat16), (((1,), (0,)), ((), ())),
            preferred_element_type=jnp.float32,
        )
        o_vmem[slot, pl.ds(k * _SUB, _SUB), :] = o + b2_ref[...]
        sub_copy(i, slot, k).start()

    @pl.when(i == _NSTEP - 1)
    def _():
        for k in range(_NSUB):
            sub_copy(i - 1, 1 - slot, k).wait()
        for k in range(_NSUB):
            sub_copy(i, slot, k).wait()


def kernel(flat, cu_seqlens, W1, b1, W2, b2):
    del cu_seqlens
    b1r = jnp.reshape(b1, (1, _HID_DIM))
    b2r = jnp.reshape(b2, (1, _NEW_DIM))
    out = pl.pallas_call(
        _mlp_tile,
        grid=(_NSTEP,),
        in_specs=[
            pl.BlockSpec(memory_space=pltpu.MemorySpace.HBM),
            pl.BlockSpec((_OLD_DIM, _HID_DIM), lambda i: (0, 0)),
            pl.BlockSpec((1, _HID_DIM), lambda i: (0, 0)),
            pl.BlockSpec((_HID_DIM, _NEW_DIM), lambda i: (0, 0)),
            pl.BlockSpec((1, _NEW_DIM), lambda i: (0, 0)),
        ],
        out_specs=pl.BlockSpec(memory_space=pltpu.MemorySpace.HBM),
        out_shape=jax.ShapeDtypeStruct((_TOTAL_TOK, _NEW_DIM), jnp.float32),
        scratch_shapes=[
            pltpu.VMEM((2, _TILE, _OLD_DIM), jnp.float32),
            pltpu.VMEM((2, _TILE, _NEW_DIM), jnp.float32),
            pltpu.SemaphoreType.DMA((2,)),
            pltpu.SemaphoreType.DMA((2,)),
        ],
        compiler_params=pltpu.CompilerParams(
            dimension_semantics=("arbitrary",),
        ),
    )(flat, W1, b1r, W2, b2r)
    return out


# speedup vs baseline: 1.1495x; 1.0644x over previous
import jax
import jax.numpy as jnp
from jax.experimental import pallas as pl
from jax.experimental.pallas import tpu as pltpu

_TOTAL_TOK = 16384
_OLD_DIM = 256
_HID_DIM = 512
_NEW_DIM = 128
_TILE = 4096
_NSTEP = _TOTAL_TOK // _TILE
_NSUB = 8
_SUB = _TILE // _NSUB


def _mlp_tile(x_ref, w1_ref, b1_ref, w2_ref, b2_ref, o_hbm, o_vmem, o_sem):
    i = pl.program_id(0)
    slot = jax.lax.rem(i, 2)

    def sub_copy(step, slot, k):
        return pltpu.make_async_copy(
            o_vmem.at[slot, pl.ds(k * _SUB, _SUB), :],
            o_hbm.at[pl.ds(step * _TILE + k * _SUB, _SUB), :],
            o_sem.at[slot],
        )

    @pl.when(i >= 2)
    def _():
        for k in range(_NSUB):
            sub_copy(i - 2, slot, k).wait()

    for k in range(_NSUB):
        x = x_ref[pl.ds(k * _SUB, _SUB), :].astype(jnp.bfloat16)
        h = jax.lax.dot_general(
            x, w1_ref[...].astype(jnp.bfloat16), (((1,), (0,)), ((), ())),
            preferred_element_type=jnp.float32,
        )
        h = jnp.maximum(
            h.astype(jnp.bfloat16) + b1_ref[...].astype(jnp.bfloat16),
            jnp.bfloat16(0.0),
        )
        o = jax.lax.dot_general(
            h, w2_ref[...].astype(jnp.bfloat16), (((1,), (0,)), ((), ())),
            preferred_element_type=jnp.float32,
        )
        o_vmem[slot, pl.ds(k * _SUB, _SUB), :] = o + b2_ref[...]
        sub_copy(i, slot, k).start()

    @pl.when(i == _NSTEP - 1)
    def _():
        for k in range(_NSUB):
            sub_copy(i - 1, 1 - slot, k).wait()
        for k in range(_NSUB):
            sub_copy(i, slot, k).wait()


def kernel(flat, cu_seqlens, W1, b1, W2, b2):
    del cu_seqlens
    b1r = jnp.reshape(b1, (1, _HID_DIM))
    b2r = jnp.reshape(b2, (1, _NEW_DIM))
    out = pl.pallas_call(
        _mlp_tile,
        grid=(_NSTEP,),
        in_specs=[
            pl.BlockSpec((_TILE, _OLD_DIM), lambda i: (i, 0)),
            pl.BlockSpec((_OLD_DIM, _HID_DIM), lambda i: (0, 0)),
            pl.BlockSpec((1, _HID_DIM), lambda i: (0, 0)),
            pl.BlockSpec((_HID_DIM, _NEW_DIM), lambda i: (0, 0)),
            pl.BlockSpec((1, _NEW_DIM), lambda i: (0, 0)),
        ],
        out_specs=pl.BlockSpec(memory_space=pltpu.MemorySpace.HBM),
        out_shape=jax.ShapeDtypeStruct((_TOTAL_TOK, _NEW_DIM), jnp.float32),
        scratch_shapes=[
            pltpu.VMEM((2, _TILE, _NEW_DIM), jnp.float32),
            pltpu.SemaphoreType.DMA((2,)),
        ],
        compiler_params=pltpu.CompilerParams(
            dimension_semantics=("arbitrary",),
        ),
    )(flat, W1, b1r, W2, b2r)
    return out
